# baseline (speedup 1.0000x reference)
import functools

import jax
import jax.numpy as jnp
from jax import lax
from jax.experimental import pallas as pl
from jax.experimental.pallas import tpu as pltpu
from jax.experimental.pallas import tpu_sc as plsc

B = 16384
SEQ = 20
D = 32
OUTW = 128
NW = 32
BPW = B // NW
GW = 128
CH = 64
NCHUNK = BPW // CH
RPC = CH * SEQ
GPC = RPC // GW

_MESH = plsc.VectorSubcoreMesh(core_axis_name="c", subcore_axis_name="s")
_NOTILE = pltpu.CompilerParams(use_tc_tiling_on_sc=False)
if "needs_layout_passes" in pltpu.CompilerParams.__dataclass_fields__:
    import dataclasses as _dc
    _NOTILE = _dc.replace(_NOTILE, needs_layout_passes=False)


def _worker_base(samples_per_worker):
    wid = lax.axis_index("s") * 2 + lax.axis_index("c")
    return wid * samples_per_worker


def _sc_text(text_table, text_idx_flat):
    @functools.partial(
        pl.kernel,
        out_type=jax.ShapeDtypeStruct((B, D), jnp.float32),
        mesh=_MESH,
        compiler_params=_NOTILE,
        scratch_types=[
            pltpu.VMEM((BPW * SEQ,), jnp.int32),
            pltpu.VMEM((RPC, D), jnp.float32),
            pltpu.VMEM((RPC, D), jnp.float32),
            pltpu.VMEM((BPW, D), jnp.float32),
            pltpu.VMEM((1, D), jnp.float32),
            pltpu.SemaphoreType.DMA,
            pltpu.SemaphoreType.DMA,
        ],
    )
    def sc_kernel(tab, idx, out, idx_v, rows_a, rows_b, tout_v, row0_v,
                  sem_a, sem_b):
        base = _worker_base(BPW)
        pltpu.sync_copy(idx.at[pl.ds(base * SEQ, BPW * SEQ)], idx_v)
        pltpu.sync_copy(tab.at[pl.ds(0, 1)], row0_v)
        lane = lax.iota(jnp.int32, 16)

        def fire(c, rows_ref, sem):
            for j in range(GPC):
                pltpu.async_copy(
                    tab.at[idx_v.at[pl.ds(c * RPC + j * GW, GW)]],
                    rows_ref.at[pl.ds(j * GW, GW)],
                    sem,
                )

        def drain(rows_ref, sem):
            pltpu.make_async_copy(tab.at[pl.ds(0, RPC)], rows_ref, sem).wait()

        def compute(c, rows_ref):
            @pl.loop(0, CH)
            def _sample(s):
                r0 = s * SEQ
                v1 = idx_v[pl.ds(c * RPC + r0, 16)]
                v2 = idx_v[pl.ds(c * RPC + r0 + 4, 16)]
                z1 = v1 == 0
                z2 = jnp.logical_and(v2 == 0, lane >= 12)
                npad = (plsc.all_reduce_population_count(z1)
                        + plsc.all_reduce_population_count(z2))
                npad_f = npad.astype(jnp.float32)
                inv = 1.0 / jnp.maximum(20.0 - npad_f, 1.0)
                for h in range(D // 16):
                    col = pl.ds(16 * h, 16)
                    acc = rows_ref[r0, col]
                    for j in range(1, SEQ):
                        acc = acc + rows_ref[r0 + j, col]
                    tout_v[c * CH + s, col] = (acc - npad_f * row0_v[0, col]) * inv

        fire(0, rows_a, sem_a)

        @pl.loop(0, NCHUNK // 2)
        def _pair(k):
            c = 2 * k
            fire(c + 1, rows_b, sem_b)
            drain(rows_a, sem_a)
            compute(c, rows_a)

            @pl.when(k < NCHUNK // 2 - 1)
            def _():
                fire(c + 2, rows_a, sem_a)

            drain(rows_b, sem_b)
            compute(c + 1, rows_b)

        pltpu.sync_copy(tout_v, out.at[pl.ds(base, BPW)])

    return sc_kernel(text_table, text_idx_flat)


def _sc_title(title_table, title_ids, text_emb):
    @functools.partial(
        pl.kernel,
        out_type=jax.ShapeDtypeStruct((B, OUTW), jnp.float32),
        mesh=_MESH,
        compiler_params=_NOTILE,
        scratch_types=[
            pltpu.VMEM((BPW,), jnp.int32),
            pltpu.VMEM((BPW, D), jnp.float32),
            pltpu.SemaphoreType.DMA,
            pltpu.SemaphoreType.DMA,
        ],
    )
    def sc_kernel(tab, idx, text, out, idx_v, rows_v, sem, sem_t):
        base = _worker_base(BPW)
        tcp = pltpu.async_copy(
            text.at[pl.ds(base, BPW)],
            out.at[pl.ds(base, BPW), pl.ds(D, D)],
            sem_t,
        )
        pltpu.sync_copy(idx.at[pl.ds(base, BPW)], idx_v)
        cps = [
            pltpu.async_copy(
                tab.at[idx_v.at[pl.ds(j * GW, GW)]],
                rows_v.at[pl.ds(j * GW, GW)],
                sem,
            )
            for j in range(BPW // GW)
        ]
        for cp in cps:
            cp.wait()
        pltpu.sync_copy(rows_v, out.at[pl.ds(base, BPW), pl.ds(0, D)])
        tcp.wait()

    return sc_kernel(title_table, title_ids, text_emb)


def kernel(title_ids, text_token_ids, title_table, text_table):
    text_emb = _sc_text(text_table, text_token_ids.reshape(-1))
    fused = _sc_title(title_table, title_ids, text_emb)
    return fused[:, : 2 * D]


# speedup vs baseline: 1.3537x; 1.3537x over previous
import functools

import jax
import jax.numpy as jnp
from jax import lax
from jax.experimental import pallas as pl
from jax.experimental.pallas import tpu as pltpu
from jax.experimental.pallas import tpu_sc as plsc

B = 16384
SEQ = 20
D = 32
OUTW = 128
NW = 32
BPW = B // NW
GW = 128
CH = 64
NCHUNK = BPW // CH
RPC = CH * SEQ
GPC = RPC // GW

_MESH = plsc.VectorSubcoreMesh(core_axis_name="c", subcore_axis_name="s")
_NOTILE = pltpu.CompilerParams(use_tc_tiling_on_sc=False)
if "needs_layout_passes" in pltpu.CompilerParams.__dataclass_fields__:
    import dataclasses as _dc
    _NOTILE = _dc.replace(_NOTILE, needs_layout_passes=False)


def _worker_base(samples_per_worker):
    wid = lax.axis_index("s") * 2 + lax.axis_index("c")
    return wid * samples_per_worker


def _sc_text(text_table, text_idx_flat):
    @functools.partial(
        pl.kernel,
        out_type=jax.ShapeDtypeStruct((B, D), jnp.float32),
        mesh=_MESH,
        compiler_params=_NOTILE,
        scratch_types=[
            pltpu.VMEM((BPW * SEQ,), jnp.int32),
            pltpu.VMEM((RPC, D), jnp.float32),
            pltpu.VMEM((RPC, D), jnp.float32),
            pltpu.VMEM((BPW, D), jnp.float32),
            pltpu.VMEM((1, D), jnp.float32),
            pltpu.SemaphoreType.DMA,
            pltpu.SemaphoreType.DMA,
        ],
    )
    def sc_kernel(tab, idx, out, idx_v, rows_a, rows_b, tout_v, row0_v,
                  sem_a, sem_b):
        base = _worker_base(BPW)
        pltpu.sync_copy(idx.at[pl.ds(base * SEQ, BPW * SEQ)], idx_v)
        pltpu.sync_copy(tab.at[pl.ds(0, 1)], row0_v)
        lane = lax.iota(jnp.int32, 16)

        def fire(c, rows_ref, sem):
            for j in range(GPC):
                pltpu.async_copy(
                    tab.at[idx_v.at[pl.ds(c * RPC + j * GW, GW)]],
                    rows_ref.at[pl.ds(j * GW, GW)],
                    sem,
                )

        def drain(rows_ref, sem):
            pltpu.make_async_copy(tab.at[pl.ds(0, RPC)], rows_ref, sem).wait()

        def compute(c, rows_ref):
            @pl.loop(0, CH, unroll=2)
            def _sample(s):
                r0 = s * SEQ
                v1 = idx_v[pl.ds(c * RPC + r0, 16)]
                v2 = idx_v[pl.ds(c * RPC + r0 + 4, 16)]
                z1 = v1 == 0
                z2 = jnp.logical_and(v2 == 0, lane >= 12)
                npad = (plsc.all_reduce_population_count(z1)
                        + plsc.all_reduce_population_count(z2))
                npad_f = npad.astype(jnp.float32)
                inv = 1.0 / jnp.maximum(20.0 - npad_f, 1.0)
                for h in range(D // 16):
                    col = pl.ds(16 * h, 16)
                    acc = rows_ref[r0, col]
                    for j in range(1, SEQ):
                        acc = acc + rows_ref[r0 + j, col]
                    tout_v[c * CH + s, col] = (acc - npad_f * row0_v[0, col]) * inv

        fire(0, rows_a, sem_a)

        @pl.loop(0, NCHUNK // 2)
        def _pair(k):
            c = 2 * k
            fire(c + 1, rows_b, sem_b)
            drain(rows_a, sem_a)
            compute(c, rows_a)

            @pl.when(k < NCHUNK // 2 - 1)
            def _():
                fire(c + 2, rows_a, sem_a)

            drain(rows_b, sem_b)
            compute(c + 1, rows_b)

        pltpu.sync_copy(tout_v, out.at[pl.ds(base, BPW)])

    return sc_kernel(text_table, text_idx_flat)


def _sc_title(title_table, title_ids):
    @functools.partial(
        pl.kernel,
        out_type=jax.ShapeDtypeStruct((B, D), jnp.float32),
        mesh=_MESH,
        compiler_params=_NOTILE,
        scratch_types=[
            pltpu.VMEM((BPW,), jnp.int32),
            pltpu.VMEM((BPW, D), jnp.float32),
            pltpu.SemaphoreType.DMA,
        ],
    )
    def sc_kernel(tab, idx, out, idx_v, rows_v, sem):
        base = _worker_base(BPW)
        pltpu.sync_copy(idx.at[pl.ds(base, BPW)], idx_v)
        cps = [
            pltpu.async_copy(
                tab.at[idx_v.at[pl.ds(j * GW, GW)]],
                rows_v.at[pl.ds(j * GW, GW)],
                sem,
            )
            for j in range(BPW // GW)
        ]
        for cp in cps:
            cp.wait()
        pltpu.sync_copy(rows_v, out.at[pl.ds(base, BPW)])

    return sc_kernel(title_table, title_ids)


def kernel(title_ids, text_token_ids, title_table, text_table):
    text_emb = _sc_text(text_table, text_token_ids.reshape(-1))
    title_emb = _sc_title(title_table, title_ids)
    return jnp.concatenate([title_emb, text_emb], axis=1)


# speedup vs baseline: 1.4280x; 1.0549x over previous
import functools

import jax
import jax.numpy as jnp
from jax import lax
from jax.experimental import pallas as pl
from jax.experimental.pallas import tpu as pltpu
from jax.experimental.pallas import tpu_sc as plsc

B = 16384
SEQ = 20
D = 32
OUTW = 128
NW = 32
BPW = B // NW
GW = 128
CH = 64
NCHUNK = BPW // CH
RPC = CH * SEQ
GPC = RPC // GW

_MESH = plsc.VectorSubcoreMesh(core_axis_name="c", subcore_axis_name="s")
_NOTILE = pltpu.CompilerParams(use_tc_tiling_on_sc=False)
if "needs_layout_passes" in pltpu.CompilerParams.__dataclass_fields__:
    import dataclasses as _dc
    _NOTILE = _dc.replace(_NOTILE, needs_layout_passes=False)


def _worker_base(samples_per_worker):
    wid = lax.axis_index("s") * 2 + lax.axis_index("c")
    return wid * samples_per_worker


def _sc_text(text_table, text_idx_flat):
    @functools.partial(
        pl.kernel,
        out_type=jax.ShapeDtypeStruct((B, D), jnp.float32),
        mesh=_MESH,
        compiler_params=_NOTILE,
        scratch_types=[
            pltpu.VMEM((BPW * SEQ,), jnp.int32),
            pltpu.VMEM((RPC, D), jnp.float32),
            pltpu.VMEM((RPC, D), jnp.float32),
            pltpu.VMEM((BPW, D), jnp.float32),
            pltpu.VMEM((1, D), jnp.float32),
            pltpu.VMEM_SHARED((10000, D), jnp.float32),
            pltpu.SemaphoreType.DMA,
            pltpu.SemaphoreType.DMA,
        ],
    )
    def sc_kernel(tab, idx, out, idx_v, rows_a, rows_b, tout_v, row0_v,
                  stab, sem_a, sem_b):
        base = _worker_base(BPW)

        @pl.when(lax.axis_index("s") == 0)
        def _stage():
            pltpu.sync_copy(tab, stab)

        pltpu.sync_copy(idx.at[pl.ds(base * SEQ, BPW * SEQ)], idx_v)
        pltpu.sync_copy(tab.at[pl.ds(0, 1)], row0_v)
        lane = lax.iota(jnp.int32, 16)
        plsc.subcore_barrier()

        def fire(c, rows_ref, sem):
            for j in range(GPC):
                pltpu.async_copy(
                    stab.at[idx_v.at[pl.ds(c * RPC + j * GW, GW)]],
                    rows_ref.at[pl.ds(j * GW, GW)],
                    sem,
                )

        def drain(rows_ref, sem):
            pltpu.make_async_copy(tab.at[pl.ds(0, RPC)], rows_ref, sem).wait()


        def compute(c, rows_ref):
            @pl.loop(0, CH, unroll=2)
            def _sample(s):
                r0 = s * SEQ
                v1 = idx_v[pl.ds(c * RPC + r0, 16)]
                v2 = idx_v[pl.ds(c * RPC + r0 + 4, 16)]
                z1 = v1 == 0
                z2 = jnp.logical_and(v2 == 0, lane >= 12)
                npad = (plsc.all_reduce_population_count(z1)
                        + plsc.all_reduce_population_count(z2))
                npad_f = npad.astype(jnp.float32)
                inv = 1.0 / jnp.maximum(20.0 - npad_f, 1.0)
                for h in range(D // 16):
                    col = pl.ds(16 * h, 16)
                    acc = rows_ref[r0, col]
                    for j in range(1, SEQ):
                        acc = acc + rows_ref[r0 + j, col]
                    tout_v[c * CH + s, col] = (acc - npad_f * row0_v[0, col]) * inv

        fire(0, rows_a, sem_a)

        @pl.loop(0, NCHUNK // 2)
        def _pair(k):
            c = 2 * k
            fire(c + 1, rows_b, sem_b)
            drain(rows_a, sem_a)
            compute(c, rows_a)

            @pl.when(k < NCHUNK // 2 - 1)
            def _():
                fire(c + 2, rows_a, sem_a)

            drain(rows_b, sem_b)
            compute(c + 1, rows_b)

        pltpu.sync_copy(tout_v, out.at[pl.ds(base, BPW)])

    return sc_kernel(text_table, text_idx_flat)


def _sc_title(title_table, title_ids):
    @functools.partial(
        pl.kernel,
        out_type=jax.ShapeDtypeStruct((B, D), jnp.float32),
        mesh=_MESH,
        compiler_params=_NOTILE,
        scratch_types=[
            pltpu.VMEM((BPW,), jnp.int32),
            pltpu.VMEM((BPW, D), jnp.float32),
            pltpu.SemaphoreType.DMA,
        ],
    )
    def sc_kernel(tab, idx, out, idx_v, rows_v, sem):
        base = _worker_base(BPW)
        pltpu.sync_copy(idx.at[pl.ds(base, BPW)], idx_v)
        cps = [
            pltpu.async_copy(
                tab.at[idx_v.at[pl.ds(j * GW, GW)]],
                rows_v.at[pl.ds(j * GW, GW)],
                sem,
            )
            for j in range(BPW // GW)
        ]
        for cp in cps:
            cp.wait()
        pltpu.sync_copy(rows_v, out.at[pl.ds(base, BPW)])

    return sc_kernel(title_table, title_ids)


def kernel(title_ids, text_token_ids, title_table, text_table):
    text_emb = _sc_text(text_table, text_token_ids.reshape(-1))
    title_emb = _sc_title(title_table, title_ids)
    return jnp.concatenate([title_emb, text_emb], axis=1)


# speedup vs baseline: 1.4763x; 1.0338x over previous
import functools

import jax
import jax.numpy as jnp
from jax import lax
from jax.experimental import pallas as pl
from jax.experimental.pallas import tpu as pltpu
from jax.experimental.pallas import tpu_sc as plsc

B = 16384
SEQ = 20
D = 32
OUTW = 128
NW = 32
BPW = B // NW
GW = 128
CH = 64
NCHUNK = BPW // CH
RPC = CH * SEQ
GPC = RPC // GW

_MESH = plsc.VectorSubcoreMesh(core_axis_name="c", subcore_axis_name="s")
_NOTILE = pltpu.CompilerParams(use_tc_tiling_on_sc=False)
if "needs_layout_passes" in pltpu.CompilerParams.__dataclass_fields__:
    import dataclasses as _dc
    _NOTILE = _dc.replace(_NOTILE, needs_layout_passes=False)


def _worker_base(samples_per_worker):
    wid = lax.axis_index("s") * 2 + lax.axis_index("c")
    return wid * samples_per_worker


def _sc_text(text_table, text_idx_flat):
    @functools.partial(
        pl.kernel,
        out_type=jax.ShapeDtypeStruct((B, D), jnp.float32),
        mesh=_MESH,
        compiler_params=_NOTILE,
        scratch_types=[
            pltpu.VMEM((BPW * SEQ,), jnp.int32),
            pltpu.VMEM((RPC, D), jnp.float32),
            pltpu.VMEM((RPC, D), jnp.float32),
            pltpu.VMEM((BPW, D), jnp.float32),
            pltpu.VMEM((1, D), jnp.float32),
            pltpu.VMEM_SHARED((10000, D), jnp.float32),
            pltpu.SemaphoreType.DMA,
            pltpu.SemaphoreType.DMA,
        ],
    )
    def sc_kernel(tab, idx, out, idx_v, rows_a, rows_b, tout_v, row0_v,
                  stab, sem_a, sem_b):
        base = _worker_base(BPW)

        @pl.when(lax.axis_index("s") == 0)
        def _stage():
            pltpu.sync_copy(tab, stab)

        pltpu.sync_copy(idx.at[pl.ds(base * SEQ, BPW * SEQ)], idx_v)
        pltpu.sync_copy(tab.at[pl.ds(0, 1)], row0_v)
        lane = lax.iota(jnp.int32, 16)
        plsc.subcore_barrier()

        def fire(c, rows_ref, sem):
            for j in range(GPC):
                pltpu.async_copy(
                    stab.at[idx_v.at[pl.ds(c * RPC + j * GW, GW)]],
                    rows_ref.at[pl.ds(j * GW, GW)],
                    sem,
                )

        def drain(rows_ref, sem):
            pltpu.make_async_copy(tab.at[pl.ds(0, RPC)], rows_ref, sem).wait()


        def compute(c, rows_ref):
            @pl.loop(0, CH, unroll=2)
            def _sample(s):
                r0 = s * SEQ
                v1 = idx_v[pl.ds(c * RPC + r0, 16)]
                v2 = idx_v[pl.ds(c * RPC + r0 + 4, 16)]
                z1 = v1 == 0
                z2 = jnp.logical_and(v2 == 0, lane >= 12)
                npad = (plsc.all_reduce_population_count(z1)
                        + plsc.all_reduce_population_count(z2))
                npad_f = npad.astype(jnp.float32)
                inv = 1.0 / jnp.maximum(20.0 - npad_f, 1.0)
                for h in range(D // 16):
                    col = pl.ds(16 * h, 16)
                    acc = rows_ref[r0, col]
                    for j in range(1, SEQ):
                        acc = acc + rows_ref[r0 + j, col]
                    tout_v[c * CH + s, col] = (acc - npad_f * row0_v[0, col]) * inv

        fire(0, rows_a, sem_a)

        @pl.loop(0, NCHUNK // 2)
        def _pair(k):
            c = 2 * k
            fire(c + 1, rows_b, sem_b)
            drain(rows_a, sem_a)
            compute(c, rows_a)

            @pl.when(k < NCHUNK // 2 - 1)
            def _():
                fire(c + 2, rows_a, sem_a)

            drain(rows_b, sem_b)
            compute(c + 1, rows_b)

        pltpu.sync_copy(tout_v, out.at[pl.ds(base, BPW)])

    return sc_kernel(text_table, text_idx_flat)


def _sc_title_merge(title_table, title_ids, text_emb):
    @functools.partial(
        pl.kernel,
        out_type=jax.ShapeDtypeStruct((B, 2 * D), jnp.float32),
        mesh=_MESH,
        compiler_params=_NOTILE,
        scratch_types=[
            pltpu.VMEM((BPW,), jnp.int32),
            pltpu.VMEM((BPW, D), jnp.float32),
            pltpu.VMEM((BPW, 2 * D), jnp.float32),
            pltpu.SemaphoreType.DMA,
            pltpu.SemaphoreType.DMA,
        ],
    )
    def sc_kernel(tab, idx, text, out, idx_v, rows_v, merge_v, sem, sem_t):
        base = _worker_base(BPW)
        tcp = pltpu.async_copy(
            text.at[pl.ds(base, BPW)],
            merge_v.at[:, pl.ds(D, D)],
            sem_t,
        )
        pltpu.sync_copy(idx.at[pl.ds(base, BPW)], idx_v)
        cps = [
            pltpu.async_copy(
                tab.at[idx_v.at[pl.ds(j * GW, GW)]],
                rows_v.at[pl.ds(j * GW, GW)],
                sem,
            )
            for j in range(BPW // GW)
        ]
        for cp in cps:
            cp.wait()
        @pl.loop(0, BPW, unroll=4)
        def _row(r):
            for h in range(D // 16):
                merge_v[r, pl.ds(16 * h, 16)] = rows_v[r, pl.ds(16 * h, 16)]

        tcp.wait()
        pltpu.sync_copy(merge_v, out.at[pl.ds(base, BPW)])

    return sc_kernel(title_table, title_ids, text_emb)


def kernel(title_ids, text_token_ids, title_table, text_table):
    text_emb = _sc_text(text_table, text_token_ids.reshape(-1))
    return _sc_title_merge(title_table, title_ids, text_emb)


# speedup vs baseline: 1.5598x; 1.0565x over previous
import functools

import jax
import jax.numpy as jnp
from jax import lax
from jax.experimental import pallas as pl
from jax.experimental.pallas import tpu as pltpu
from jax.experimental.pallas import tpu_sc as plsc

B = 16384
SEQ = 20
D = 32
OUTW = 128
NW = 32
BPW = B // NW
GW = 128
CH = 64
NCHUNK = BPW // CH
RPC = CH * SEQ
GPC = RPC // GW

_MESH = plsc.VectorSubcoreMesh(core_axis_name="c", subcore_axis_name="s")
_NOTILE = pltpu.CompilerParams(use_tc_tiling_on_sc=False)
if "needs_layout_passes" in pltpu.CompilerParams.__dataclass_fields__:
    import dataclasses as _dc
    _NOTILE = _dc.replace(_NOTILE, needs_layout_passes=False)


def _worker_base(samples_per_worker):
    wid = lax.axis_index("s") * 2 + lax.axis_index("c")
    return wid * samples_per_worker


def _sc_text(text_table, text_idx_flat):
    @functools.partial(
        pl.kernel,
        out_type=jax.ShapeDtypeStruct((B, D), jnp.float32),
        mesh=_MESH,
        compiler_params=_NOTILE,
        scratch_types=[
            pltpu.VMEM((BPW * SEQ,), jnp.int32),
            pltpu.VMEM((RPC, D), jnp.float32),
            pltpu.VMEM((RPC, D), jnp.float32),
            pltpu.VMEM((BPW, D), jnp.float32),
            pltpu.VMEM((1, D), jnp.float32),
            pltpu.VMEM_SHARED((10000, D), jnp.float32),
            pltpu.SemaphoreType.DMA,
            pltpu.SemaphoreType.DMA,
        ],
    )
    def sc_kernel(tab, idx, out, idx_v, rows_a, rows_b, tout_v, row0_v,
                  stab, sem_a, sem_b):
        base = _worker_base(BPW)

        @pl.when(lax.axis_index("s") == 0)
        def _stage():
            pltpu.sync_copy(tab, stab)

        pltpu.sync_copy(idx.at[pl.ds(base * SEQ, BPW * SEQ)], idx_v)
        pltpu.sync_copy(tab.at[pl.ds(0, 1)], row0_v)
        lane = lax.iota(jnp.int32, 16)
        plsc.subcore_barrier()

        def fire(c, rows_ref, sem):
            for j in range(GPC):
                pltpu.async_copy(
                    stab.at[idx_v.at[pl.ds(c * RPC + j * GW, GW)]],
                    rows_ref.at[pl.ds(j * GW, GW)],
                    sem,
                )

        def drain(rows_ref, sem):
            pltpu.make_async_copy(tab.at[pl.ds(0, RPC)], rows_ref, sem).wait()


        def compute(c, rows_ref):
            return

            @pl.loop(0, CH, unroll=2)
            def _sample(s):
                r0 = s * SEQ
                v1 = idx_v[pl.ds(c * RPC + r0, 16)]
                v2 = idx_v[pl.ds(c * RPC + r0 + 4, 16)]
                z1 = v1 == 0
                z2 = jnp.logical_and(v2 == 0, lane >= 12)
                npad = (plsc.all_reduce_population_count(z1)
                        + plsc.all_reduce_population_count(z2))
                npad_f = npad.astype(jnp.float32)
                inv = 1.0 / jnp.maximum(20.0 - npad_f, 1.0)
                for h in range(D // 16):
                    col = pl.ds(16 * h, 16)
                    acc = rows_ref[r0, col]
                    for j in range(1, SEQ):
                        acc = acc + rows_ref[r0 + j, col]
                    tout_v[c * CH + s, col] = (acc - npad_f * row0_v[0, col]) * inv

        fire(0, rows_a, sem_a)

        @pl.loop(0, NCHUNK // 2)
        def _pair(k):
            c = 2 * k
            fire(c + 1, rows_b, sem_b)
            drain(rows_a, sem_a)
            compute(c, rows_a)

            @pl.when(k < NCHUNK // 2 - 1)
            def _():
                fire(c + 2, rows_a, sem_a)

            drain(rows_b, sem_b)
            compute(c + 1, rows_b)

        pltpu.sync_copy(tout_v, out.at[pl.ds(base, BPW)])

    return sc_kernel(text_table, text_idx_flat)


def _sc_title_merge(title_table, title_ids, text_emb):
    @functools.partial(
        pl.kernel,
        out_type=jax.ShapeDtypeStruct((B, 2 * D), jnp.float32),
        mesh=_MESH,
        compiler_params=_NOTILE,
        scratch_types=[
            pltpu.VMEM((BPW,), jnp.int32),
            pltpu.VMEM((BPW, D), jnp.float32),
            pltpu.VMEM((BPW, 2 * D), jnp.float32),
            pltpu.SemaphoreType.DMA,
            pltpu.SemaphoreType.DMA,
        ],
    )
    def sc_kernel(tab, idx, text, out, idx_v, rows_v, merge_v, sem, sem_t):
        base = _worker_base(BPW)
        tcp = pltpu.async_copy(
            text.at[pl.ds(base, BPW)],
            merge_v.at[:, pl.ds(D, D)],
            sem_t,
        )
        pltpu.sync_copy(idx.at[pl.ds(base, BPW)], idx_v)
        cps = [
            pltpu.async_copy(
                tab.at[idx_v.at[pl.ds(j * GW, GW)]],
                rows_v.at[pl.ds(j * GW, GW)],
                sem,
            )
            for j in range(BPW // GW)
        ]
        for cp in cps:
            cp.wait()
        @pl.loop(0, BPW, unroll=4)
        def _row(r):
            for h in range(D // 16):
                merge_v[r, pl.ds(16 * h, 16)] = rows_v[r, pl.ds(16 * h, 16)]

        tcp.wait()
        pltpu.sync_copy(merge_v, out.at[pl.ds(base, BPW)])

    return sc_kernel(title_table, title_ids, text_emb)


def kernel(title_ids, text_token_ids, title_table, text_table):
    text_emb = _sc_text(text_table, text_token_ids.reshape(-1))
    return _sc_title_merge(title_table, title_ids, text_emb)


# speedup vs baseline: 1.5633x; 1.0023x over previous
import functools

import jax
import jax.numpy as jnp
from jax import lax
from jax.experimental import pallas as pl
from jax.experimental.pallas import tpu as pltpu
from jax.experimental.pallas import tpu_sc as plsc

B = 16384
SEQ = 20
D = 32
OUTW = 128
NW = 32
BPW = B // NW
GW = 128
CH = 64
NCHUNK = BPW // CH
RPC = CH * SEQ
GPC = RPC // GW

_MESH = plsc.VectorSubcoreMesh(core_axis_name="c", subcore_axis_name="s")
_NOTILE = pltpu.CompilerParams(use_tc_tiling_on_sc=False)
if "needs_layout_passes" in pltpu.CompilerParams.__dataclass_fields__:
    import dataclasses as _dc
    _NOTILE = _dc.replace(_NOTILE, needs_layout_passes=False)


def _worker_base(samples_per_worker):
    wid = lax.axis_index("s") * 2 + lax.axis_index("c")
    return wid * samples_per_worker


def _sc_text(text_table, text_idx_flat):
    @functools.partial(
        pl.kernel,
        out_type=jax.ShapeDtypeStruct((B, D), jnp.float32),
        mesh=_MESH,
        compiler_params=_NOTILE,
        scratch_types=[
            pltpu.VMEM((BPW * SEQ,), jnp.int32),
            pltpu.VMEM((RPC, D), jnp.float32),
            pltpu.VMEM((RPC, D), jnp.float32),
            pltpu.VMEM((BPW, D), jnp.float32),
            pltpu.VMEM((1, D), jnp.float32),
            pltpu.VMEM_SHARED((10000, D), jnp.float32),
            pltpu.SemaphoreType.DMA,
            pltpu.SemaphoreType.DMA,
        ],
    )
    def sc_kernel(tab, idx, out, idx_v, rows_a, rows_b, tout_v, row0_v,
                  stab, sem_a, sem_b):
        base = _worker_base(BPW)

        @pl.when(lax.axis_index("s") == 0)
        def _stage():
            pltpu.sync_copy(tab, stab)

        pltpu.sync_copy(idx.at[pl.ds(base * SEQ, BPW * SEQ)], idx_v)
        pltpu.sync_copy(tab.at[pl.ds(0, 1)], row0_v)
        lane = lax.iota(jnp.int32, 16)
        plsc.subcore_barrier()

        def fire(c, rows_ref, sem):
            for j in range(GPC):
                pltpu.async_copy(
                    stab.at[idx_v.at[pl.ds(c * RPC + j * GW, GW)]],
                    rows_ref.at[pl.ds(j * GW, GW)],
                    sem,
                )

        def drain(rows_ref, sem):
            pltpu.make_async_copy(tab.at[pl.ds(0, RPC)], rows_ref, sem).wait()


        def compute(c, rows_ref):
            @pl.loop(0, CH, unroll=2)
            def _sample(s):
                r0 = s * SEQ
                v1 = idx_v[pl.ds(c * RPC + r0, 16)]
                v2 = idx_v[pl.ds(c * RPC + r0 + 4, 16)]
                z1 = v1 == 0
                z2 = jnp.logical_and(v2 == 0, lane >= 12)
                npad = (plsc.all_reduce_population_count(z1)
                        + plsc.all_reduce_population_count(z2))
                npad_f = npad.astype(jnp.float32)
                inv = 1.0 / jnp.maximum(20.0 - npad_f, 1.0)
                for h in range(D // 16):
                    col = pl.ds(16 * h, 16)
                    acc = rows_ref[r0, col]
                    for j in range(1, SEQ):
                        acc = acc + rows_ref[r0 + j, col]
                    tout_v[c * CH + s, col] = (acc - npad_f * row0_v[0, col]) * inv

        fire(0, rows_a, sem_a)

        @pl.loop(0, NCHUNK // 2)
        def _pair(k):
            c = 2 * k
            fire(c + 1, rows_b, sem_b)
            drain(rows_a, sem_a)
            compute(c, rows_a)

            @pl.when(k < NCHUNK // 2 - 1)
            def _():
                fire(c + 2, rows_a, sem_a)

            drain(rows_b, sem_b)
            compute(c + 1, rows_b)

        pltpu.sync_copy(tout_v, out.at[pl.ds(base, BPW)])

    return sc_kernel(text_table, text_idx_flat)


def _sc_title_merge(title_table, title_ids, text_emb):
    @functools.partial(
        pl.kernel,
        out_type=jax.ShapeDtypeStruct((B, OUTW), jnp.float32),
        mesh=_MESH,
        compiler_params=_NOTILE,
        scratch_types=[
            pltpu.VMEM((BPW,), jnp.int32),
            pltpu.VMEM((BPW, D), jnp.float32),
            pltpu.VMEM((BPW, OUTW), jnp.float32),
            pltpu.SemaphoreType.DMA,
            pltpu.SemaphoreType.DMA,
        ],
    )
    def sc_kernel(tab, idx, text, out, idx_v, rows_v, merge_v, sem, sem_t):
        base = _worker_base(BPW)
        tcp = pltpu.async_copy(
            text.at[pl.ds(base, BPW)],
            merge_v.at[:, pl.ds(D, D)],
            sem_t,
        )
        pltpu.sync_copy(idx.at[pl.ds(base, BPW)], idx_v)
        cps = [
            pltpu.async_copy(
                tab.at[idx_v.at[pl.ds(j * GW, GW)]],
                rows_v.at[pl.ds(j * GW, GW)],
                sem,
            )
            for j in range(BPW // GW)
        ]
        for cp in cps:
            cp.wait()
        @pl.loop(0, BPW, unroll=4)
        def _row(r):
            for h in range(D // 16):
                merge_v[r, pl.ds(16 * h, 16)] = rows_v[r, pl.ds(16 * h, 16)]

        tcp.wait()
        pltpu.sync_copy(merge_v, out.at[pl.ds(base, BPW)])

    return sc_kernel(title_table, title_ids, text_emb)


def kernel(title_ids, text_token_ids, title_table, text_table):
    text_emb = _sc_text(text_table, text_token_ids.reshape(-1))
    fused = _sc_title_merge(title_table, title_ids, text_emb)
    return fused[:, : 2 * D]


# speedup vs baseline: 1.6623x; 1.0634x over previous
import functools

import jax
import jax.numpy as jnp
from jax import lax
from jax.experimental import pallas as pl
from jax.experimental.pallas import tpu as pltpu
from jax.experimental.pallas import tpu_sc as plsc

B = 16384
SEQ = 20
D = 32
OUTW = 128
NW = 32
BPW = B // NW
GW = 128
CH = 64
NCHUNK = BPW // CH
RPC = CH * SEQ
GPC = RPC // GW

_MESH = plsc.VectorSubcoreMesh(core_axis_name="c", subcore_axis_name="s")
_NOTILE = pltpu.CompilerParams(use_tc_tiling_on_sc=False)
if "needs_layout_passes" in pltpu.CompilerParams.__dataclass_fields__:
    import dataclasses as _dc
    _NOTILE = _dc.replace(_NOTILE, needs_layout_passes=False)


def _worker_base(samples_per_worker):
    wid = lax.axis_index("s") * 2 + lax.axis_index("c")
    return wid * samples_per_worker


def _sc_text(text_table, text_idx_flat):
    @functools.partial(
        pl.kernel,
        out_type=jax.ShapeDtypeStruct((B, D), jnp.float32),
        mesh=_MESH,
        compiler_params=_NOTILE,
        scratch_types=[
            pltpu.VMEM((BPW * SEQ,), jnp.int32),
            pltpu.VMEM((RPC, D), jnp.bfloat16),
            pltpu.VMEM((RPC, D), jnp.bfloat16),
            pltpu.VMEM((BPW, D), jnp.float32),
            pltpu.VMEM((1, D), jnp.bfloat16),
            pltpu.VMEM_SHARED((10000, D), jnp.bfloat16),
            pltpu.SemaphoreType.DMA,
            pltpu.SemaphoreType.DMA,
        ],
    )
    def sc_kernel(tab, idx, out, idx_v, rows_a, rows_b, tout_v, row0_v,
                  stab, sem_a, sem_b):
        base = _worker_base(BPW)

        @pl.when(lax.axis_index("s") == 0)
        def _stage():
            pltpu.sync_copy(tab, stab)

        pltpu.sync_copy(idx.at[pl.ds(base * SEQ, BPW * SEQ)], idx_v)
        pltpu.sync_copy(tab.at[pl.ds(0, 1)], row0_v)
        lane = lax.iota(jnp.int32, 16)
        plsc.subcore_barrier()
        r0lo, r0hi = plsc.unpack(row0_v[0, pl.ds(0, D)],
                                 format=plsc.PackFormat.INTERLEAVED)

        def fire(c, rows_ref, sem):
            for j in range(GPC):
                pltpu.async_copy(
                    stab.at[idx_v.at[pl.ds(c * RPC + j * GW, GW)]],
                    rows_ref.at[pl.ds(j * GW, GW)],
                    sem,
                )

        def drain(rows_ref, sem):
            pltpu.make_async_copy(tab.at[pl.ds(0, RPC)], rows_ref, sem).wait()


        def compute(c, rows_ref):
            @pl.loop(0, CH, unroll=2)
            def _sample(s):
                r0 = s * SEQ
                v1 = idx_v[pl.ds(c * RPC + r0, 16)]
                v2 = idx_v[pl.ds(c * RPC + r0 + 4, 16)]
                z1 = v1 == 0
                z2 = jnp.logical_and(v2 == 0, lane >= 12)
                npad = (plsc.all_reduce_population_count(z1)
                        + plsc.all_reduce_population_count(z2))
                npad_f = npad.astype(jnp.float32)
                inv = 1.0 / jnp.maximum(20.0 - npad_f, 1.0)
                lo, hi = plsc.unpack(rows_ref[r0, pl.ds(0, D)],
                                     format=plsc.PackFormat.INTERLEAVED)
                for j in range(1, SEQ):
                    lo_j, hi_j = plsc.unpack(rows_ref[r0 + j, pl.ds(0, D)],
                                             format=plsc.PackFormat.INTERLEAVED)
                    lo = lo + lo_j
                    hi = hi + hi_j
                tout_v[c * CH + s, pl.ds(0, 16)] = (lo - npad_f * r0lo) * inv
                tout_v[c * CH + s, pl.ds(16, 16)] = (hi - npad_f * r0hi) * inv

        fire(0, rows_a, sem_a)

        @pl.loop(0, NCHUNK // 2)
        def _pair(k):
            c = 2 * k
            fire(c + 1, rows_b, sem_b)
            drain(rows_a, sem_a)
            compute(c, rows_a)

            @pl.when(k < NCHUNK // 2 - 1)
            def _():
                fire(c + 2, rows_a, sem_a)

            drain(rows_b, sem_b)
            compute(c + 1, rows_b)

        pltpu.sync_copy(tout_v, out.at[pl.ds(base, BPW)])

    return sc_kernel(text_table, text_idx_flat)


def _sc_title_merge(title_table, title_ids, text_emb):
    @functools.partial(
        pl.kernel,
        out_type=jax.ShapeDtypeStruct((B, OUTW), jnp.float32),
        mesh=_MESH,
        compiler_params=_NOTILE,
        scratch_types=[
            pltpu.VMEM((BPW,), jnp.int32),
            pltpu.VMEM((BPW, D), jnp.float32),
            pltpu.VMEM((BPW, OUTW), jnp.float32),
            pltpu.SemaphoreType.DMA,
            pltpu.SemaphoreType.DMA,
        ],
    )
    def sc_kernel(tab, idx, text, out, idx_v, rows_v, merge_v, sem, sem_t):
        base = _worker_base(BPW)
        tcp = pltpu.async_copy(
            text.at[pl.ds(base, BPW)],
            merge_v.at[:, pl.ds(D, D)],
            sem_t,
        )
        pltpu.sync_copy(idx.at[pl.ds(base, BPW)], idx_v)
        cps = [
            pltpu.async_copy(
                tab.at[idx_v.at[pl.ds(j * GW, GW)]],
                rows_v.at[pl.ds(j * GW, GW)],
                sem,
            )
            for j in range(BPW // GW)
        ]
        for cp in cps:
            cp.wait()
        @pl.loop(0, BPW, unroll=4)
        def _row(r):
            for h in range(D // 16):
                merge_v[r, pl.ds(16 * h, 16)] = rows_v[r, pl.ds(16 * h, 16)]

        tcp.wait()
        pltpu.sync_copy(merge_v, out.at[pl.ds(base, BPW)])

    return sc_kernel(title_table, title_ids, text_emb)


def kernel(title_ids, text_token_ids, title_table, text_table):
    tperm = jnp.stack(
        [text_table[:, :16], text_table[:, 16:]], axis=2
    ).reshape(10000, D).astype(jnp.bfloat16)
    text_emb = _sc_text(tperm, text_token_ids.reshape(-1))
    fused = _sc_title_merge(title_table, title_ids, text_emb)
    return fused[:, : 2 * D]


# speedup vs baseline: 1.6661x; 1.0023x over previous
import functools

import jax
import jax.numpy as jnp
from jax import lax
from jax.experimental import pallas as pl
from jax.experimental.pallas import tpu as pltpu
from jax.experimental.pallas import tpu_sc as plsc

B = 16384
SEQ = 20
D = 32
OUTW = 128
NW = 32
BPW = B // NW
GW = 128
CH = 128
NCHUNK = BPW // CH
RPC = CH * SEQ
GPC = RPC // GW

_MESH = plsc.VectorSubcoreMesh(core_axis_name="c", subcore_axis_name="s")
_NOTILE = pltpu.CompilerParams(use_tc_tiling_on_sc=False)
if "needs_layout_passes" in pltpu.CompilerParams.__dataclass_fields__:
    import dataclasses as _dc
    _NOTILE = _dc.replace(_NOTILE, needs_layout_passes=False)


def _worker_base(samples_per_worker):
    wid = lax.axis_index("s") * 2 + lax.axis_index("c")
    return wid * samples_per_worker


def _sc_text(text_table, text_idx_flat):
    @functools.partial(
        pl.kernel,
        out_type=jax.ShapeDtypeStruct((B, D), jnp.float32),
        mesh=_MESH,
        compiler_params=_NOTILE,
        scratch_types=[
            pltpu.VMEM((BPW * SEQ,), jnp.int32),
            pltpu.VMEM((RPC, D), jnp.bfloat16),
            pltpu.VMEM((RPC, D), jnp.bfloat16),
            pltpu.VMEM((BPW, D), jnp.float32),
            pltpu.VMEM((1, D), jnp.bfloat16),
            pltpu.VMEM_SHARED((10000, D), jnp.bfloat16),
            pltpu.SemaphoreType.DMA,
            pltpu.SemaphoreType.DMA,
        ],
    )
    def sc_kernel(tab, idx, out, idx_v, rows_a, rows_b, tout_v, row0_v,
                  stab, sem_a, sem_b):
        base = _worker_base(BPW)

        @pl.when(lax.axis_index("s") == 0)
        def _stage():
            pltpu.sync_copy(tab, stab)

        pltpu.sync_copy(idx.at[pl.ds(base * SEQ, BPW * SEQ)], idx_v)
        pltpu.sync_copy(tab.at[pl.ds(0, 1)], row0_v)
        lane = lax.iota(jnp.int32, 16)
        plsc.subcore_barrier()
        r0lo, r0hi = plsc.unpack(row0_v[0, pl.ds(0, D)],
                                 format=plsc.PackFormat.INTERLEAVED)

        def fire(c, rows_ref, sem):
            for j in range(GPC):
                pltpu.async_copy(
                    stab.at[idx_v.at[pl.ds(c * RPC + j * GW, GW)]],
                    rows_ref.at[pl.ds(j * GW, GW)],
                    sem,
                )

        def drain(rows_ref, sem):
            pltpu.make_async_copy(tab.at[pl.ds(0, RPC)], rows_ref, sem).wait()


        def compute(c, rows_ref):
            @pl.loop(0, CH, unroll=2)
            def _sample(s):
                r0 = s * SEQ
                v1 = idx_v[pl.ds(c * RPC + r0, 16)]
                v2 = idx_v[pl.ds(c * RPC + r0 + 4, 16)]
                z1 = v1 == 0
                z2 = jnp.logical_and(v2 == 0, lane >= 12)
                npad = (plsc.all_reduce_population_count(z1)
                        + plsc.all_reduce_population_count(z2))
                npad_f = npad.astype(jnp.float32)
                inv = 1.0 / jnp.maximum(20.0 - npad_f, 1.0)
                lo, hi = plsc.unpack(rows_ref[r0, pl.ds(0, D)],
                                     format=plsc.PackFormat.INTERLEAVED)
                for j in range(1, SEQ):
                    lo_j, hi_j = plsc.unpack(rows_ref[r0 + j, pl.ds(0, D)],
                                             format=plsc.PackFormat.INTERLEAVED)
                    lo = lo + lo_j
                    hi = hi + hi_j
                tout_v[c * CH + s, pl.ds(0, 16)] = (lo - npad_f * r0lo) * inv
                tout_v[c * CH + s, pl.ds(16, 16)] = (hi - npad_f * r0hi) * inv

        fire(0, rows_a, sem_a)

        @pl.loop(0, NCHUNK // 2)
        def _pair(k):
            c = 2 * k
            fire(c + 1, rows_b, sem_b)
            drain(rows_a, sem_a)
            compute(c, rows_a)

            @pl.when(k < NCHUNK // 2 - 1)
            def _():
                fire(c + 2, rows_a, sem_a)

            drain(rows_b, sem_b)
            compute(c + 1, rows_b)

        pltpu.sync_copy(tout_v, out.at[pl.ds(base, BPW)])

    return sc_kernel(text_table, text_idx_flat)


def _sc_title_merge(title_table, title_ids, text_emb):
    @functools.partial(
        pl.kernel,
        out_type=jax.ShapeDtypeStruct((B, OUTW), jnp.float32),
        mesh=_MESH,
        compiler_params=_NOTILE,
        scratch_types=[
            pltpu.VMEM((BPW,), jnp.int32),
            pltpu.VMEM((BPW, D), jnp.float32),
            pltpu.VMEM((BPW, OUTW), jnp.float32),
            pltpu.SemaphoreType.DMA,
            pltpu.SemaphoreType.DMA,
        ],
    )
    def sc_kernel(tab, idx, text, out, idx_v, rows_v, merge_v, sem, sem_t):
        base = _worker_base(BPW)
        tcp = pltpu.async_copy(
            text.at[pl.ds(base, BPW)],
            merge_v.at[:, pl.ds(D, D)],
            sem_t,
        )
        pltpu.sync_copy(idx.at[pl.ds(base, BPW)], idx_v)
        cps = [
            pltpu.async_copy(
                tab.at[idx_v.at[pl.ds(j * GW, GW)]],
                rows_v.at[pl.ds(j * GW, GW)],
                sem,
            )
            for j in range(BPW // GW)
        ]
        for cp in cps:
            cp.wait()
        @pl.loop(0, BPW, unroll=4)
        def _row(r):
            for h in range(D // 16):
                merge_v[r, pl.ds(16 * h, 16)] = rows_v[r, pl.ds(16 * h, 16)]

        tcp.wait()
        pltpu.sync_copy(merge_v, out.at[pl.ds(base, BPW)])

    return sc_kernel(title_table, title_ids, text_emb)


def kernel(title_ids, text_token_ids, title_table, text_table):
    tperm = jnp.stack(
        [text_table[:, :16], text_table[:, 16:]], axis=2
    ).reshape(10000, D).astype(jnp.bfloat16)
    text_emb = _sc_text(tperm, text_token_ids.reshape(-1))
    fused = _sc_title_merge(title_table, title_ids, text_emb)
    return fused[:, : 2 * D]
